# Initial kernel scaffold; baseline (speedup 1.0000x reference)
import functools

import jax
import jax.numpy as jnp
from jax import lax
from jax.experimental import pallas as pl
from jax.experimental.pallas import tpu as pltpu
from jax.experimental.pallas import tpu_sc as plsc

VOCAB = 1_000_000
EMB = 64
RANK = 1
WORLD = 4
LOCAL_N = VOCAB // WORLD
OFFSET = RANK * LOCAL_N
BATCH = 4096
SEQ = 50
TOTAL = BATCH * SEQ

NC = 2
NS = 16
NW = NC * NS
L = 16

PER_W = TOTAL // NW
CHUNK = 640
NCHUNK = PER_W // CHUNK
SUB = CHUNK // 128


def _tec_body(ids_hbm, table_hbm, out_hbm, ids_v, gidx_v, maskf_v, rows_v,
              gsem, ssem):
    wid = lax.axis_index("s") * NC + lax.axis_index("c")
    base = wid * PER_W

    pltpu.sync_copy(ids_hbm.at[pl.ds(base, PER_W)], ids_v)

    def compute_chunk(g):
        b = g % 2

        def jbody(j, _):
            v = ids_v[pl.ds(g * CHUNK + 16 * j, 16)]
            adj = v - OFFSET
            valid = (adj >= 0) & (adj < LOCAL_N)
            gidx_v[b, j // 8, pl.ds((j % 8) * 16, 16)] = jnp.where(valid, adj, 0)
            maskf_v[b, pl.ds(16 * j, 16)] = valid.astype(jnp.float32)
            return 0

        lax.fori_loop(0, CHUNK // 16, jbody, 0, unroll=8)

    def fire_gather(g):
        b = g % 2
        handles = []
        for s in range(SUB):
            handles.append(pltpu.async_copy(
                table_hbm.at[gidx_v.at[b, s]],
                rows_v.at[b, pl.ds(s * 128, 128), :],
                gsem))
        return handles

    def mask_mul(g):
        b = g % 2

        def ibody(i, _):
            m = maskf_v[b, i]
            for k in range(4):
                r = rows_v[b, i, pl.ds(16 * k, 16)]
                rows_v[b, i, pl.ds(16 * k, 16)] = r * m
            return 0

        lax.fori_loop(0, CHUNK, ibody, 0, unroll=8)

    def fire_scatter(g):
        b = g % 2
        return pltpu.async_copy(
            rows_v.at[b],
            out_hbm.at[pl.ds(base + g * CHUNK, CHUNK), :],
            ssem)

    compute_chunk(0)
    gh = fire_gather(0)
    sh = [None] * NCHUNK
    for g in range(NCHUNK):
        if g + 1 < NCHUNK:
            compute_chunk(g + 1)
            if g >= 1:
                sh[g - 1].wait()
            gh_next = fire_gather(g + 1)
        for h in gh:
            h.wait()
        if g + 1 < NCHUNK:
            gh = gh_next
        mask_mul(g)
        sh[g] = fire_scatter(g)
    sh[NCHUNK - 2].wait()
    sh[NCHUNK - 1].wait()


@jax.jit
def _embed(ids_flat, weights):
    kern = pl.kernel(
        _tec_body,
        out_type=jax.ShapeDtypeStruct((TOTAL, EMB), jnp.float32),
        mesh=plsc.VectorSubcoreMesh(core_axis_name="c", subcore_axis_name="s"),
        scratch_types=[
            pltpu.VMEM((PER_W,), jnp.int32),
            pltpu.VMEM((2, SUB, 128), jnp.int32),
            pltpu.VMEM((2, CHUNK), jnp.float32),
            pltpu.VMEM((2, CHUNK, EMB), jnp.float32),
            pltpu.SemaphoreType.DMA,
            pltpu.SemaphoreType.DMA,
        ],
    )
    return kern(ids_flat, weights)


def kernel(input_ids, weights):
    out = _embed(input_ids.reshape(TOTAL), weights)
    return out.reshape(BATCH, SEQ, EMB)


# speedup vs baseline: 1.9734x; 1.9734x over previous
import functools

import jax
import jax.numpy as jnp
from jax import lax
from jax.experimental import pallas as pl
from jax.experimental.pallas import tpu as pltpu
from jax.experimental.pallas import tpu_sc as plsc

VOCAB = 1_000_000
EMB = 64
RANK = 1
WORLD = 4
LOCAL_N = VOCAB // WORLD
OFFSET = RANK * LOCAL_N
BATCH = 4096
SEQ = 50
TOTAL = BATCH * SEQ

NC = 2
NS = 16
NW = NC * NS
L = 16

PER_W = TOTAL // NW
CHUNK = 640
NCHUNK = PER_W // CHUNK
SUB = CHUNK // 128


def _tec_body(ids_hbm, table_hbm, out_hbm, ids_v, gidx_v, maskf_v, rows_v,
              gsem, ssem):
    wid = lax.axis_index("s") * NC + lax.axis_index("c")
    base = wid * PER_W

    pltpu.sync_copy(ids_hbm.at[pl.ds(base, PER_W)], ids_v)

    def compute_chunk(g):
        b = g % 2

        zi = jnp.zeros((16,), jnp.int32)
        zf = jnp.zeros((16,), jnp.float32)
        of = jnp.ones((16,), jnp.float32)

        def jbody(j, _):
            v = ids_v[pl.ds(g * CHUNK + 16 * j, 16)]
            adj = v - OFFSET
            valid = (adj >= 0) & (adj < LOCAL_N)
            gidx_v[b, j // 8, pl.ds((j % 8) * 16, 16)] = jnp.where(valid, adj, zi)
            maskf_v[b, pl.ds(16 * j, 16)] = jnp.where(valid, of, zf)
            return 0

        lax.fori_loop(0, CHUNK // 16, jbody, 0, unroll=8)

    def fire_gather(g):
        b = g % 2
        handles = []
        for s in range(SUB):
            handles.append(pltpu.async_copy(
                table_hbm.at[gidx_v.at[b, s]],
                rows_v.at[b, pl.ds(s * 128, 128), :],
                gsem))
        return handles

    def mask_mul(g):
        b = g % 2

        def jbody(j, _):
            mvec = maskf_v[b, pl.ds(16 * j, 16)]
            for jj in range(16):
                m = mvec[jj]
                row = 16 * j + jj
                for k in range(4):
                    r = rows_v[b, row, pl.ds(16 * k, 16)]
                    rows_v[b, row, pl.ds(16 * k, 16)] = r * m
            return 0

        lax.fori_loop(0, CHUNK // 16, jbody, 0)

    def fire_scatter(g):
        b = g % 2
        return pltpu.async_copy(
            rows_v.at[b],
            out_hbm.at[pl.ds(base + g * CHUNK, CHUNK), :],
            ssem)

    compute_chunk(0)
    gh = fire_gather(0)
    sh = [None] * NCHUNK
    for g in range(NCHUNK):
        if g + 1 < NCHUNK:
            compute_chunk(g + 1)
            if g >= 1:
                sh[g - 1].wait()
            gh_next = fire_gather(g + 1)
        for h in gh:
            h.wait()
        if g + 1 < NCHUNK:
            gh = gh_next
        mask_mul(g)
        sh[g] = fire_scatter(g)
    sh[NCHUNK - 2].wait()
    sh[NCHUNK - 1].wait()


@jax.jit
def _embed(ids_flat, weights):
    kern = pl.kernel(
        _tec_body,
        out_type=jax.ShapeDtypeStruct((TOTAL, EMB), jnp.float32),
        mesh=plsc.VectorSubcoreMesh(core_axis_name="c", subcore_axis_name="s"),
        scratch_types=[
            pltpu.VMEM((PER_W,), jnp.int32),
            pltpu.VMEM((2, SUB, 128), jnp.int32),
            pltpu.VMEM((2, CHUNK), jnp.float32),
            pltpu.VMEM((2, CHUNK, EMB), jnp.float32),
            pltpu.SemaphoreType.DMA,
            pltpu.SemaphoreType.DMA,
        ],
        compiler_params=pltpu.CompilerParams(use_tc_tiling_on_sc=False),
    )
    return kern(ids_flat, weights)


def kernel(input_ids, weights):
    out = _embed(input_ids.reshape(TOTAL), weights)
    return out.reshape(BATCH, SEQ, EMB)


# speedup vs baseline: 14.1934x; 7.1923x over previous
import jax
import jax.numpy as jnp
from jax import lax
from jax.experimental import pallas as pl
from jax.experimental.pallas import tpu as pltpu
from jax.experimental.pallas import tpu_sc as plsc

VOCAB = 1_000_000
EMB = 64
RANK = 1
WORLD = 4
LOCAL_N = VOCAB // WORLD
OFFSET = RANK * LOCAL_N
BATCH = 4096
SEQ = 50
TOTAL = BATCH * SEQ

NC = 2
NS = 16
NW = NC * NS

PER_W = TOTAL // NW
CHUNK = 640
NCHUNK = PER_W // CHUNK
SG = 32
MAXS = CHUNK // SG
NGRP = CHUNK // 16


def _tec_body(ids_hbm, table_hbm, out_hbm, ids_v, cid_flat, cid2d, maskf_v,
              psrc_v, gbuf, gsem, ssem):
    wid = lax.axis_index("s") * NC + lax.axis_index("c")
    base = wid * PER_W

    zi = jnp.zeros((16,), jnp.int32)
    zf = jnp.zeros((16,), jnp.float32)
    of = jnp.ones((16,), jnp.float32)

    pltpu.sync_copy(ids_hbm.at[pl.ds(base, PER_W)], ids_v)

    for t in range(0, 2 * (CHUNK + 16), 16):
        cid_flat[pl.ds(t, 16)] = zi
    for b in range(2):
        for k in range(4):
            gbuf[b, 0, pl.ds(16 * k, 16)] = zf

    def compute_chunk(g):
        b = g % 2

        def jbody(j, cnt):
            v = ids_v[pl.ds(g * CHUNK + 16 * j, 16)]
            adj = v - OFFSET
            valid = (adj >= 0) & (adj < LOCAL_N)
            maskf_v[b, pl.ds(16 * j, 16)] = jnp.where(valid, of, zf)
            vi = jnp.where(valid, jnp.full((16,), 1, jnp.int32), zi)
            incl = plsc.cumsum(vi)
            psrc_v[b, pl.ds(16 * j, 16)] = (incl - vi) + cnt
            plsc.store_compressed(
                cid_flat.at[pl.ds(b * (CHUNK + 16) + cnt, 16)], adj,
                mask=valid)
            return cnt + incl[15]

        return lax.fori_loop(0, NGRP, jbody, jnp.int32(0))

    def fire_gather(g, cnt):
        b = g % 2
        ns = (cnt + (SG - 1)) // SG

        def sbody(s, _):
            fb = b * (CHUNK + 16)
            cid2d[b, s, pl.ds(0, 16)] = cid_flat[pl.ds(fb + s * SG, 16)]
            cid2d[b, s, pl.ds(16, 16)] = cid_flat[pl.ds(fb + s * SG + 16, 16)]
            pltpu.async_copy(table_hbm.at[cid2d.at[b, s]],
                             gbuf.at[b, pl.ds(s * SG, SG), :], gsem)
            return 0

        lax.fori_loop(0, ns, sbody, 0)

    def wait_gather(g, cnt):
        b = g % 2
        ns = (cnt + (SG - 1)) // SG

        def wbody(s, _):
            pltpu.make_async_copy(table_hbm.at[cid2d.at[b, 0]],
                                  gbuf.at[b, pl.ds(0, SG), :], gsem).wait()
            return 0

        lax.fori_loop(0, ns, wbody, 0)

    def expand(g):
        b = g % 2

        def jbody(jr, _):
            j = (NGRP - 1) - jr
            mvec = maskf_v[b, pl.ds(16 * j, 16)]
            svec = psrc_v[b, pl.ds(16 * j, 16)]
            for jj in range(15, -1, -1):
                m = mvec[jj]
                srow = svec[jj]
                drow = 16 * j + jj
                for k in range(4):
                    r = gbuf[b, srow, pl.ds(16 * k, 16)]
                    gbuf[b, drow, pl.ds(16 * k, 16)] = r * m
            return 0

        lax.fori_loop(0, NGRP, jbody, 0)

    def fire_scatter(g):
        b = g % 2
        return pltpu.async_copy(
            gbuf.at[b],
            out_hbm.at[pl.ds(base + g * CHUNK, CHUNK), :],
            ssem)

    cnts = [None] * NCHUNK
    sh = [None] * NCHUNK
    cnts[0] = compute_chunk(0)
    fire_gather(0, cnts[0])
    for g in range(NCHUNK):
        if g + 1 < NCHUNK:
            cnts[g + 1] = compute_chunk(g + 1)
            if g >= 1:
                sh[g - 1].wait()
            fire_gather(g + 1, cnts[g + 1])
        wait_gather(g, cnts[g])
        expand(g)
        sh[g] = fire_scatter(g)
    sh[NCHUNK - 2].wait()
    sh[NCHUNK - 1].wait()


@jax.jit
def _embed(ids_flat, weights):
    kern = pl.kernel(
        _tec_body,
        out_type=jax.ShapeDtypeStruct((TOTAL, EMB), jnp.float32),
        mesh=plsc.VectorSubcoreMesh(core_axis_name="c", subcore_axis_name="s"),
        scratch_types=[
            pltpu.VMEM((PER_W,), jnp.int32),
            pltpu.VMEM((2 * (CHUNK + 16),), jnp.int32),
            pltpu.VMEM((2, MAXS, SG), jnp.int32),
            pltpu.VMEM((2, CHUNK), jnp.float32),
            pltpu.VMEM((2, CHUNK), jnp.int32),
            pltpu.VMEM((2, CHUNK, EMB), jnp.float32),
            pltpu.SemaphoreType.DMA,
            pltpu.SemaphoreType.DMA,
        ],
        compiler_params=pltpu.CompilerParams(use_tc_tiling_on_sc=False,
                                             needs_layout_passes=False),
    )
    return kern(ids_flat, weights)


def kernel(input_ids, weights):
    out = _embed(input_ids.reshape(TOTAL), weights)
    return out.reshape(BATCH, SEQ, EMB)


# speedup vs baseline: 14.5633x; 1.0261x over previous
import jax
import jax.numpy as jnp
from jax import lax
from jax.experimental import pallas as pl
from jax.experimental.pallas import tpu as pltpu
from jax.experimental.pallas import tpu_sc as plsc

VOCAB = 1_000_000
EMB = 64
RANK = 1
WORLD = 4
LOCAL_N = VOCAB // WORLD
OFFSET = RANK * LOCAL_N
BATCH = 4096
SEQ = 50
TOTAL = BATCH * SEQ

NC = 2
NS = 16
NW = NC * NS

PER_W = TOTAL // NW
CHUNK = 640
NCHUNK = PER_W // CHUNK
SG = 16
MAXS = CHUNK // SG
NGRP = CHUNK // 16


def _tec_body(ids_hbm, table_hbm, out_hbm, ids_v, cid_flat, cid2d, maskf_v,
              psrc_v, gbuf, gsem, ssem):
    wid = lax.axis_index("s") * NC + lax.axis_index("c")
    base = wid * PER_W

    zi = jnp.zeros((16,), jnp.int32)
    zf = jnp.zeros((16,), jnp.float32)
    of = jnp.ones((16,), jnp.float32)

    pltpu.sync_copy(ids_hbm.at[pl.ds(base, PER_W)], ids_v)

    for t in range(0, 2 * (CHUNK + 16), 16):
        cid_flat[pl.ds(t, 16)] = zi
    for b in range(2):
        for k in range(4):
            gbuf[b, 0, pl.ds(16 * k, 16)] = zf

    def compute_chunk(g):
        b = g % 2

        def jbody(j, cnt):
            v = ids_v[pl.ds(g * CHUNK + 16 * j, 16)]
            adj = v - OFFSET
            valid = (adj >= 0) & (adj < LOCAL_N)
            maskf_v[b, pl.ds(16 * j, 16)] = jnp.where(valid, of, zf)
            vi = jnp.where(valid, jnp.full((16,), 1, jnp.int32), zi)
            incl = plsc.cumsum(vi)
            psrc_v[b, pl.ds(16 * j, 16)] = (incl - vi) + cnt
            plsc.store_compressed(
                cid_flat.at[pl.ds(b * (CHUNK + 16) + cnt, 16)], adj,
                mask=valid)
            return cnt + incl[15]

        return lax.fori_loop(0, NGRP, jbody, jnp.int32(0))

    def fire_gather(g, cnt):
        b = g % 2
        ns = (cnt + (SG - 1)) // SG

        def sbody(s, _):
            fb = b * (CHUNK + 16)
            cid2d[b, s, pl.ds(0, 16)] = cid_flat[pl.ds(fb + s * SG, 16)]
            pltpu.async_copy(table_hbm.at[cid2d.at[b, s]],
                             gbuf.at[b, pl.ds(s * SG, SG), :], gsem)
            return 0

        lax.fori_loop(0, ns, sbody, 0)

    def wait_gather(g, cnt):
        b = g % 2
        ns = (cnt + (SG - 1)) // SG

        def wbody(s, _):
            pltpu.make_async_copy(table_hbm.at[cid2d.at[b, 0]],
                                  gbuf.at[b, pl.ds(0, SG), :], gsem).wait()
            return 0

        lax.fori_loop(0, ns, wbody, 0)

    def expand(g):
        b = g % 2

        def jbody(jr, _):
            j = (NGRP - 1) - jr
            mvec = maskf_v[b, pl.ds(16 * j, 16)]
            svec = psrc_v[b, pl.ds(16 * j, 16)]
            for jj in range(15, -1, -1):
                m = mvec[jj]
                srow = svec[jj]
                drow = 16 * j + jj
                for k in range(4):
                    r = gbuf[b, srow, pl.ds(16 * k, 16)]
                    gbuf[b, drow, pl.ds(16 * k, 16)] = r * m
            return 0

        lax.fori_loop(0, NGRP, jbody, 0)

    def fire_scatter(g):
        b = g % 2
        return pltpu.async_copy(
            gbuf.at[b],
            out_hbm.at[pl.ds(base + g * CHUNK, CHUNK), :],
            ssem)

    cnts = [None] * NCHUNK
    sh = [None] * NCHUNK
    cnts[0] = compute_chunk(0)
    fire_gather(0, cnts[0])
    for g in range(NCHUNK):
        if g + 1 < NCHUNK:
            cnts[g + 1] = compute_chunk(g + 1)
            if g >= 1:
                sh[g - 1].wait()
            fire_gather(g + 1, cnts[g + 1])
        wait_gather(g, cnts[g])
        expand(g)
        sh[g] = fire_scatter(g)
    sh[NCHUNK - 2].wait()
    sh[NCHUNK - 1].wait()


@jax.jit
def _embed(ids_flat, weights):
    kern = pl.kernel(
        _tec_body,
        out_type=jax.ShapeDtypeStruct((TOTAL, EMB), jnp.float32),
        mesh=plsc.VectorSubcoreMesh(core_axis_name="c", subcore_axis_name="s"),
        scratch_types=[
            pltpu.VMEM((PER_W,), jnp.int32),
            pltpu.VMEM((2 * (CHUNK + 16),), jnp.int32),
            pltpu.VMEM((2, MAXS, SG), jnp.int32),
            pltpu.VMEM((2, CHUNK), jnp.float32),
            pltpu.VMEM((2, CHUNK), jnp.int32),
            pltpu.VMEM((2, CHUNK, EMB), jnp.float32),
            pltpu.SemaphoreType.DMA,
            pltpu.SemaphoreType.DMA,
        ],
        compiler_params=pltpu.CompilerParams(use_tc_tiling_on_sc=False,
                                             needs_layout_passes=False),
    )
    return kern(ids_flat, weights)


def kernel(input_ids, weights):
    out = _embed(input_ids.reshape(TOTAL), weights)
    return out.reshape(BATCH, SEQ, EMB)
